# baseline (speedup 1.0000x reference)
import functools

import jax
import jax.numpy as jnp
from jax import lax
from jax.experimental import pallas as pl
from jax.experimental.pallas import tpu as pltpu
from jax.experimental.pallas import tpu_sc as plsc

_NC = 2
_NS = 16
_NW = _NC * _NS


def _rne16(u):
    t = (u >> 16) & jnp.int32(0xFFFF)
    r = (u & jnp.int32(0xFFFF)) + jnp.int32(0x7FFF) + ((u >> 16) & jnp.int32(1))
    return (t + ((r >> 16) & jnp.int32(1))) & jnp.int32(0xFFFF)


def _tc_pack(fm):
    N, F = fm.shape
    H = F // 2
    RB = 2000

    def body(x_ref, out_ref):
        u = lax.bitcast_convert_type(x_ref[...], jnp.int32)
        lo = _rne16(u[:, :H])
        hi = _rne16(u[:, H:])
        out_ref[...] = lo | (hi << 16)

    return pl.pallas_call(
        body,
        grid=(N // RB,),
        in_specs=[pl.BlockSpec((RB, F), lambda i: (i, 0))],
        out_specs=pl.BlockSpec((RB, H), lambda i: (i, 0)),
        out_shape=jax.ShapeDtypeStruct((N, H), jnp.int32),
    )(fm)


def _sc_gather_all(forest0, forest1f, forest2f, fmp):
    B = forest0.shape[0]
    N1 = forest1f.size
    N2 = forest2f.size
    PW = fmp.shape[1]
    F = 2 * PW
    S2 = N2 // N1
    n0 = B // _NW
    n1 = N1 // _NW
    IR = forest2f.shape[1]
    CH = 16
    ROWS = CH * S2
    nchunk1 = n1 // IR
    nseg = (N2 // S2) // _NW
    nchunk2 = nseg // CH

    mesh = plsc.VectorSubcoreMesh(core_axis_name="c", subcore_axis_name="s")

    @functools.partial(
        pl.kernel,
        mesh=mesh,
        out_type=[
            jax.ShapeDtypeStruct((B, PW), jnp.int32),
            jax.ShapeDtypeStruct((N1, PW), jnp.int32),
            jax.ShapeDtypeStruct((N1, F), jnp.float32),
        ],
        scratch_types=[
            pltpu.VMEM((n0,), jnp.int32),
            pltpu.VMEM((nchunk1, IR), jnp.int32),
            pltpu.VMEM((2 * nchunk2, IR), jnp.int32),
            pltpu.VMEM((ROWS, PW), jnp.int32),
            pltpu.VMEM((ROWS, PW), jnp.int32),
            pltpu.VMEM((CH, F), jnp.float32),
            pltpu.VMEM((CH, F), jnp.float32),
            pltpu.SemaphoreType.DMA,
            pltpu.SemaphoreType.DMA,
            pltpu.SemaphoreType.DMA,
            pltpu.SemaphoreType.DMA,
            pltpu.SemaphoreType.DMA,
            pltpu.SemaphoreType.DMA,
            pltpu.SemaphoreType.DMA,
        ],
    )
    def sc_kernel(f0_hbm, f1_hbm, f2_hbm, fm_hbm, out0, out1, out2,
                  idx0_v, idx1_v, idx2_v, bufa_v, bufb_v, acca_v, accb_v,
                  sema, semb, semwa, semwb, sem0, semw1a, semw1b):
        wid = lax.axis_index("s") * _NC + lax.axis_index("c")

        pltpu.sync_copy(f1_hbm.at[pl.ds(wid * nchunk1, nchunk1)], idx1_v)
        pltpu.sync_copy(f2_hbm.at[pl.ds(wid * 2 * nchunk2, 2 * nchunk2)],
                        idx2_v)

        base0 = wid * n0
        pltpu.sync_copy(f0_hbm.at[pl.ds(base0, n0)], idx0_v)
        pltpu.async_copy(fm_hbm.at[idx0_v], bufa_v.at[pl.ds(0, n0)], sem0).wait()
        pltpu.sync_copy(bufa_v.at[pl.ds(0, n0)], out0.at[pl.ds(base0, n0)])

        def f1_issue(c, buf_v, sem):
            @pl.when(c < nchunk1)
            def _():
                pltpu.async_copy(fm_hbm.at[idx1_v.at[c]],
                                 buf_v.at[pl.ds(0, IR)], sem)

        def f1_wait_write(buf_v, semw):
            pltpu.make_async_copy(buf_v.at[pl.ds(0, IR)],
                                  out1.at[pl.ds(wid * n1, IR)], semw).wait()

        def f1_drain(c, buf_v, sem, semw):
            pltpu.make_async_copy(fm_hbm.at[idx1_v.at[c]],
                                  buf_v.at[pl.ds(0, IR)], sem).wait()
            pltpu.async_copy(buf_v.at[pl.ds(0, IR)],
                             out1.at[pl.ds(wid * n1 + c * IR, IR)], semw)

        f1_issue(0, bufa_v, sema)
        f1_issue(1, bufb_v, semb)
        f1_drain(0, bufa_v, sema, semw1a)
        f1_drain(1, bufb_v, semb, semw1b)
        for c in range(2, nchunk1):
            buf_v = bufa_v if c % 2 == 0 else bufb_v
            sem = sema if c % 2 == 0 else semb
            semw = semw1a if c % 2 == 0 else semw1b
            f1_wait_write(buf_v, semw)
            f1_issue(c, buf_v, sem)
            f1_drain(c, buf_v, sem, semw)
        f1_wait_write(bufa_v, semw1a)
        f1_wait_write(bufb_v, semw1b)

        def x2_issue(c, buf_v, sem):
            @pl.when(c < nchunk2)
            def _():
                pltpu.async_copy(fm_hbm.at[idx2_v.at[2 * c]],
                                 buf_v.at[pl.ds(0, IR)], sem)
                pltpu.async_copy(fm_hbm.at[idx2_v.at[2 * c + 1]],
                                 buf_v.at[pl.ds(IR, IR)], sem)

        def x2_wait_write(acc_v, semw):
            pltpu.make_async_copy(
                acc_v, out2.at[pl.ds(wid * nseg, CH)], semw).wait()

        def x2_drain(c, buf_v, sem, acc_v, semw, wait_prev):
            pltpu.make_async_copy(fm_hbm.at[idx2_v.at[2 * c]],
                                  buf_v.at[pl.ds(0, IR)], sem).wait()
            pltpu.make_async_copy(fm_hbm.at[idx2_v.at[2 * c + 1]],
                                  buf_v.at[pl.ds(IR, IR)], sem).wait()
            if wait_prev:
                x2_wait_write(acc_v, semw)

            def seg(s, inner):
                r0 = s * S2
                for kk in range(PW // 16):
                    col = kk * 16
                    w = buf_v[r0, pl.ds(col, 16)]
                    alo = lax.bitcast_convert_type(w << 16, jnp.float32)
                    ahi = lax.bitcast_convert_type(
                        w & jnp.int32(-65536), jnp.float32)
                    for j in range(1, S2):
                        w = buf_v[r0 + j, pl.ds(col, 16)]
                        alo = alo + lax.bitcast_convert_type(w << 16, jnp.float32)
                        ahi = ahi + lax.bitcast_convert_type(
                            w & jnp.int32(-65536), jnp.float32)
                    acc_v[s, pl.ds(col, 16)] = alo
                    acc_v[s, pl.ds(PW + col, 16)] = ahi
                return inner
            lax.fori_loop(0, CH, seg, 0)
            pltpu.async_copy(acc_v, out2.at[pl.ds(wid * nseg + c * CH, CH)],
                             semw)

        def x2_pair_body(c0, wait_prev):
            x2_issue(c0 + 1, bufb_v, semb)
            x2_drain(c0, bufa_v, sema, acca_v, semwa, wait_prev)
            x2_issue(c0 + 2, bufa_v, sema)
            x2_drain(c0 + 1, bufb_v, semb, accb_v, semwb, wait_prev)

        x2_issue(0, bufa_v, sema)
        x2_pair_body(0, False)

        def x2_pair(p, carry):
            x2_pair_body(2 * p, True)
            return carry
        lax.fori_loop(1, nchunk2 // 2, x2_pair, 0)

        x2_wait_write(acca_v, semwa)
        x2_wait_write(accb_v, semwb)

    return sc_kernel(forest0, forest1f, forest2f, fmp)


def _unpack_feat(packed_i32):
    flo = lax.bitcast_convert_type(packed_i32 << 16, jnp.float32)
    fhi = lax.bitcast_convert_type(packed_i32 & jnp.int32(-65536), jnp.float32)
    return flo, fhi


def _tc_layers(feat1p, x2s, feat0p, walo, wahi, wb16lo, wb16hi, w2a, w2b16):
    N1, PW = feat1p.shape
    F = x2s.shape[1]
    B = feat0p.shape[0]
    R = 2048
    G = R // 16
    grid = N1 // R

    def bdot(a, b_ref):
        return jnp.dot(a.astype(jnp.bfloat16), b_ref[...],
                       preferred_element_type=jnp.float32)

    def body(f1_ref, x2_ref, f0_ref, walo_ref, wahi_ref, wblo_ref, wbhi_ref,
             w2a_ref, w2b_ref, out_ref, h0_s, h1s_s):
        i = pl.program_id(0)
        flo, fhi = _unpack_feat(f1_ref[...])
        x2 = x2_ref[...]
        h1 = bdot(flo, walo_ref)
        h1 = h1 + bdot(fhi, wahi_ref)
        h1 = h1 + bdot(x2[:, :PW], wblo_ref)
        h1 = h1 + bdot(x2[:, PW:], wbhi_ref)
        h1 = jnp.maximum(h1, 0.0)
        h1s_s[pl.ds(i * G, G), :] = h1.reshape(G, 16, F).sum(axis=1)
        xlo = flo.reshape(G, 16, PW).sum(axis=1)
        xhi = fhi.reshape(G, 16, PW).sum(axis=1)
        f0lo, f0hi = _unpack_feat(f0_ref[...])
        h0 = bdot(f0lo, walo_ref)
        h0 = h0 + bdot(f0hi, wahi_ref)
        h0 = h0 + bdot(xlo, wblo_ref)
        h0 = h0 + bdot(xhi, wbhi_ref)
        h0_s[pl.ds(i * G, G), :] = jnp.maximum(h0, 0.0)

        @pl.when(i == grid - 1)
        def _():
            o = bdot(h0_s[...], w2a_ref)
            o = o + bdot(h1s_s[...], w2b_ref)
            out_ref[...] = jnp.maximum(o, 0.0)

    return pl.pallas_call(
        body,
        grid=(grid,),
        in_specs=[
            pl.BlockSpec((R, PW), lambda i: (i, 0)),
            pl.BlockSpec((R, F), lambda i: (i, 0)),
            pl.BlockSpec((G, PW), lambda i: (i, 0)),
            pl.BlockSpec((PW, F), lambda i: (0, 0)),
            pl.BlockSpec((PW, F), lambda i: (0, 0)),
            pl.BlockSpec((PW, F), lambda i: (0, 0)),
            pl.BlockSpec((PW, F), lambda i: (0, 0)),
            pl.BlockSpec((F, F), lambda i: (0, 0)),
            pl.BlockSpec((F, F), lambda i: (0, 0)),
        ],
        out_specs=pl.BlockSpec((B, F), lambda i: (0, 0)),
        out_shape=jax.ShapeDtypeStruct((B, F), jnp.float32),
        scratch_shapes=[
            pltpu.VMEM((B, F), jnp.float32),
            pltpu.VMEM((B, F), jnp.float32),
        ],
    )(feat1p, x2s, feat0p, walo, wahi, wb16lo, wb16hi, w2a, w2b16)


def kernel(forest0, forest1, forest2, feature_matrix, W1, W2):
    N, F = feature_matrix.shape
    H = F // 2
    f0 = forest0.astype(jnp.int32)
    f1 = forest1.reshape(-1).astype(jnp.int32)
    f2 = forest2.reshape(-1).astype(jnp.int32)

    fmp = _tc_pack(feature_matrix)

    feat0p, feat1p, x2s = _sc_gather_all(f0, f1.reshape(-1, 128), f2.reshape(-1, 128), fmp)

    W1t = W1.T
    w1a = W1t[:F]
    w1b16 = W1t[F:] * (1.0 / 16.0)
    walo, wahi = w1a[:H], w1a[H:]
    wb16lo, wb16hi = w1b16[:H], w1b16[H:]

    W2t = W2.T
    w2a = W2t[:F]
    w2b16 = W2t[F:] * (1.0 / 16.0)

    bf = jnp.bfloat16
    return _tc_layers(feat1p, x2s, feat0p,
                      walo.astype(bf), wahi.astype(bf),
                      wb16lo.astype(bf), wb16hi.astype(bf),
                      w2a.astype(bf), w2b16.astype(bf))


# speedup vs baseline: 1.0071x; 1.0071x over previous
import functools

import jax
import jax.numpy as jnp
from jax import lax
from jax.experimental import pallas as pl
from jax.experimental.pallas import tpu as pltpu
from jax.experimental.pallas import tpu_sc as plsc

_NC = 2
_NS = 16
_NW = _NC * _NS


def _rne16(u):
    t = (u >> 16) & jnp.int32(0xFFFF)
    r = (u & jnp.int32(0xFFFF)) + jnp.int32(0x7FFF) + ((u >> 16) & jnp.int32(1))
    return (t + ((r >> 16) & jnp.int32(1))) & jnp.int32(0xFFFF)


def _tc_pack(fm):
    N, F = fm.shape
    H = F // 2
    RB = 2000

    def body(x_ref, out_ref):
        u = lax.bitcast_convert_type(x_ref[...], jnp.int32)
        lo = _rne16(u[:, :H])
        hi = _rne16(u[:, H:])
        out_ref[...] = lo | (hi << 16)

    return pl.pallas_call(
        body,
        grid=(N // RB,),
        in_specs=[pl.BlockSpec((RB, F), lambda i: (i, 0))],
        out_specs=pl.BlockSpec((RB, H), lambda i: (i, 0)),
        out_shape=jax.ShapeDtypeStruct((N, H), jnp.int32),
    )(fm)


def _sc_gather_all(forest0, forest1f, forest2f, fmp):
    B = forest0.shape[0]
    N1 = forest1f.size
    N2 = forest2f.size
    PW = fmp.shape[1]
    F = 2 * PW
    S2 = N2 // N1
    n0 = B // _NW
    n1 = N1 // _NW
    CH = 8
    ROWS = CH * S2
    nchunk1 = n1 // ROWS
    nseg = (N2 // S2) // _NW
    nchunk2 = nseg // CH

    mesh = plsc.VectorSubcoreMesh(core_axis_name="c", subcore_axis_name="s")

    @functools.partial(
        pl.kernel,
        mesh=mesh,
        out_type=[
            jax.ShapeDtypeStruct((B, PW), jnp.int32),
            jax.ShapeDtypeStruct((N1, PW), jnp.int32),
            jax.ShapeDtypeStruct((N1, F), jnp.float32),
        ],
        scratch_types=[
            pltpu.VMEM((n0,), jnp.int32),
            pltpu.VMEM((nchunk1, ROWS), jnp.int32),
            pltpu.VMEM((nchunk2, ROWS), jnp.int32),
            pltpu.VMEM((ROWS, PW), jnp.int32),
            pltpu.VMEM((ROWS, PW), jnp.int32),
            pltpu.VMEM((ROWS, PW), jnp.int32),
            pltpu.VMEM((ROWS, PW), jnp.int32),
            pltpu.VMEM((ROWS, PW), jnp.int32),
            pltpu.VMEM((ROWS, PW), jnp.int32),
            pltpu.VMEM((CH, F), jnp.float32),
            pltpu.VMEM((CH, F), jnp.float32),
            pltpu.VMEM((CH, F), jnp.float32),
            pltpu.VMEM((CH, F), jnp.float32),
            pltpu.SemaphoreType.DMA,
            pltpu.SemaphoreType.DMA,
            pltpu.SemaphoreType.DMA,
            pltpu.SemaphoreType.DMA,
            pltpu.SemaphoreType.DMA,
            pltpu.SemaphoreType.DMA,
            pltpu.SemaphoreType.DMA,
            pltpu.SemaphoreType.DMA,
            pltpu.SemaphoreType.DMA,
            pltpu.SemaphoreType.DMA,
            pltpu.SemaphoreType.DMA,
            pltpu.SemaphoreType.DMA,
        ],
    )
    def sc_kernel(f0_hbm, f1_hbm, f2_hbm, fm_hbm, out0, out1, out2,
                  idx0_v, idx1_v, idx2_v, bufa_v, bufb_v, bufc_v, bufd_v,
                  bufe_v, buff_v, acca_v, accb_v, accc_v, accd_v,
                  sema, semb, semc, semd, seme, semf,
                  semwa, semwb, semwc, semwd, semwe, semwf):
        wid = lax.axis_index("s") * _NC + lax.axis_index("c")

        pltpu.sync_copy(f1_hbm.at[pl.ds(wid * nchunk1, nchunk1)], idx1_v)
        pltpu.sync_copy(f2_hbm.at[pl.ds(wid * nchunk2, nchunk2)], idx2_v)

        def x2_issue(c, buf_v, sem):
            @pl.when(c < nchunk2)
            def _():
                pltpu.async_copy(fm_hbm.at[idx2_v.at[c]], buf_v, sem)

        x2_issue(0, bufa_v, sema)
        x2_issue(1, bufb_v, semb)
        x2_issue(2, bufc_v, semc)

        base0 = wid * n0
        pltpu.sync_copy(f0_hbm.at[pl.ds(base0, n0)], idx0_v)
        pltpu.async_copy(fm_hbm.at[idx0_v], bufe_v.at[pl.ds(0, n0)], seme).wait()
        pltpu.sync_copy(bufe_v.at[pl.ds(0, n0)], out0.at[pl.ds(base0, n0)])

        def f1_issue(c, buf_v, sem):
            @pl.when(c < nchunk1)
            def _():
                pltpu.async_copy(fm_hbm.at[idx1_v.at[c]], buf_v, sem)

        def f1_wait_write(buf_v, semw):
            pltpu.make_async_copy(
                buf_v, out1.at[pl.ds(wid * n1, ROWS)], semw).wait()

        def f1_drain(c, buf_v, sem, semw, first):
            pltpu.make_async_copy(fm_hbm.at[idx1_v.at[c]], buf_v, sem).wait()
            pltpu.async_copy(buf_v, out1.at[pl.ds(wid * n1 + c * ROWS, ROWS)],
                             semw)

        f1_issue(0, bufe_v, seme)
        f1_issue(1, buff_v, semf)
        f1_drain(0, bufe_v, seme, semwe, True)
        f1_drain(1, buff_v, semf, semwf, True)
        for c in range(2, nchunk1):
            buf_v = bufe_v if c % 2 == 0 else buff_v
            sem = seme if c % 2 == 0 else semf
            semw = semwe if c % 2 == 0 else semwf
            f1_wait_write(buf_v, semw)
            f1_issue(c, buf_v, sem)
            f1_drain(c, buf_v, sem, semw, False)
        f1_wait_write(bufe_v, semwe)
        f1_wait_write(buff_v, semwf)


        def x2_wait_write(acc_v, semw):
            pltpu.make_async_copy(
                acc_v, out2.at[pl.ds(wid * nseg, CH)], semw).wait()

        def x2_drain(c, buf_v, sem, acc_v, semw, wait_prev):
            pltpu.make_async_copy(fm_hbm.at[idx2_v.at[c]], buf_v, sem).wait()
            if wait_prev:
                x2_wait_write(acc_v, semw)

            def seg(s, inner):
                r0 = s * S2
                for kk in range(PW // 16):
                    col = kk * 16
                    w = buf_v[r0, pl.ds(col, 16)]
                    alo = lax.bitcast_convert_type(w << 16, jnp.float32)
                    ahi = lax.bitcast_convert_type(
                        w & jnp.int32(-65536), jnp.float32)
                    for j in range(1, S2):
                        w = buf_v[r0 + j, pl.ds(col, 16)]
                        alo = alo + lax.bitcast_convert_type(w << 16, jnp.float32)
                        ahi = ahi + lax.bitcast_convert_type(
                            w & jnp.int32(-65536), jnp.float32)
                    acc_v[s, pl.ds(col, 16)] = alo
                    acc_v[s, pl.ds(PW + col, 16)] = ahi
                return inner
            lax.fori_loop(0, CH, seg, 0)
            pltpu.async_copy(acc_v, out2.at[pl.ds(wid * nseg + c * CH, CH)],
                             semw)

        def x2_quad_body(p, c0, wait_prev):
            x2_issue(c0 + 3, bufd_v, semd)
            x2_drain(c0, bufa_v, sema, acca_v, semwa, wait_prev)
            x2_issue(c0 + 4, bufa_v, sema)
            x2_drain(c0 + 1, bufb_v, semb, accb_v, semwb, wait_prev)
            x2_issue(c0 + 5, bufb_v, semb)
            x2_drain(c0 + 2, bufc_v, semc, accc_v, semwc, wait_prev)
            x2_issue(c0 + 6, bufc_v, semc)
            x2_drain(c0 + 3, bufd_v, semd, accd_v, semwd, wait_prev)

        x2_quad_body(0, 0, False)

        def x2_quad(p, carry):
            x2_quad_body(p, 4 * p, True)
            return carry
        lax.fori_loop(1, nchunk2 // 4, x2_quad, 0)

        x2_wait_write(acca_v, semwa)
        x2_wait_write(accb_v, semwb)
        x2_wait_write(accc_v, semwc)
        x2_wait_write(accd_v, semwd)

    return sc_kernel(forest0, forest1f, forest2f, fmp)


def _unpack_feat(packed_i32):
    flo = lax.bitcast_convert_type(packed_i32 << 16, jnp.float32)
    fhi = lax.bitcast_convert_type(packed_i32 & jnp.int32(-65536), jnp.float32)
    return flo, fhi


def _tc_layers(feat1p, x2s, feat0p, walo, wahi, wb16lo, wb16hi, w2a, w2b16):
    N1, PW = feat1p.shape
    F = x2s.shape[1]
    B = feat0p.shape[0]
    R = 2048
    G = R // 16
    grid = N1 // R

    def bdot(a, b_ref):
        return jnp.dot(a.astype(jnp.bfloat16), b_ref[...],
                       preferred_element_type=jnp.float32)

    def body(f1_ref, x2_ref, f0_ref, walo_ref, wahi_ref, wblo_ref, wbhi_ref,
             w2a_ref, w2b_ref, out_ref, h0_s, h1s_s):
        i = pl.program_id(0)
        flo, fhi = _unpack_feat(f1_ref[...])
        x2 = x2_ref[...]
        h1 = bdot(flo, walo_ref)
        h1 = h1 + bdot(fhi, wahi_ref)
        h1 = h1 + bdot(x2[:, :PW], wblo_ref)
        h1 = h1 + bdot(x2[:, PW:], wbhi_ref)
        h1 = jnp.maximum(h1, 0.0)
        h1s_s[pl.ds(i * G, G), :] = h1.reshape(G, 16, F).sum(axis=1)
        xlo = flo.reshape(G, 16, PW).sum(axis=1)
        xhi = fhi.reshape(G, 16, PW).sum(axis=1)
        f0lo, f0hi = _unpack_feat(f0_ref[...])
        h0 = bdot(f0lo, walo_ref)
        h0 = h0 + bdot(f0hi, wahi_ref)
        h0 = h0 + bdot(xlo, wblo_ref)
        h0 = h0 + bdot(xhi, wbhi_ref)
        h0_s[pl.ds(i * G, G), :] = jnp.maximum(h0, 0.0)

        @pl.when(i == grid - 1)
        def _():
            o = bdot(h0_s[...], w2a_ref)
            o = o + bdot(h1s_s[...], w2b_ref)
            out_ref[...] = jnp.maximum(o, 0.0)

    return pl.pallas_call(
        body,
        grid=(grid,),
        in_specs=[
            pl.BlockSpec((R, PW), lambda i: (i, 0)),
            pl.BlockSpec((R, F), lambda i: (i, 0)),
            pl.BlockSpec((G, PW), lambda i: (i, 0)),
            pl.BlockSpec((PW, F), lambda i: (0, 0)),
            pl.BlockSpec((PW, F), lambda i: (0, 0)),
            pl.BlockSpec((PW, F), lambda i: (0, 0)),
            pl.BlockSpec((PW, F), lambda i: (0, 0)),
            pl.BlockSpec((F, F), lambda i: (0, 0)),
            pl.BlockSpec((F, F), lambda i: (0, 0)),
        ],
        out_specs=pl.BlockSpec((B, F), lambda i: (0, 0)),
        out_shape=jax.ShapeDtypeStruct((B, F), jnp.float32),
        scratch_shapes=[
            pltpu.VMEM((B, F), jnp.float32),
            pltpu.VMEM((B, F), jnp.float32),
        ],
    )(feat1p, x2s, feat0p, walo, wahi, wb16lo, wb16hi, w2a, w2b16)


def kernel(forest0, forest1, forest2, feature_matrix, W1, W2):
    N, F = feature_matrix.shape
    H = F // 2
    f0 = forest0.astype(jnp.int32)
    f1 = forest1.reshape(-1).astype(jnp.int32)
    f2 = forest2.reshape(-1).astype(jnp.int32)

    fmp = _tc_pack(feature_matrix)

    feat0p, feat1p, x2s = _sc_gather_all(f0, f1.reshape(-1, 128), f2.reshape(-1, 128), fmp)

    W1t = W1.T
    w1a = W1t[:F]
    w1b16 = W1t[F:] * (1.0 / 16.0)
    walo, wahi = w1a[:H], w1a[H:]
    wb16lo, wb16hi = w1b16[:H], w1b16[H:]

    W2t = W2.T
    w2a = W2t[:F]
    w2b16 = W2t[F:] * (1.0 / 16.0)

    bf = jnp.bfloat16
    return _tc_layers(feat1p, x2s, feat0p,
                      walo.astype(bf), wahi.astype(bf),
                      wb16lo.astype(bf), wb16hi.astype(bf),
                      w2a.astype(bf), w2b16.astype(bf))
